# baseline (speedup 1.0000x reference)
import functools

import jax
import jax.numpy as jnp
from jax import lax
from jax.experimental import pallas as pl
from jax.experimental.pallas import tpu as pltpu
from jax.experimental.pallas import tpu_sc as plsc

BATCH = 16384
EMBED_DIM = 32
HIDDEN_DIM = 64
_PACK = 4
_PROW = _PACK * EMBED_DIM

_NUM_CORES = 2
_NUM_SUBCORES = 16
_NW = _NUM_CORES * _NUM_SUBCORES
_B_PER_W = BATCH // _NW
_CHUNK = 128
_NCHUNK = _B_PER_W // _CHUNK


def _gather_body(uid_hbm, iid_hbm, up_hbm, ip_hbm, out_u, out_i,
                 idx_v, rows4, sem, isem):
    wid = lax.axis_index("s") * _NUM_CORES + lax.axis_index("c")
    base = wid * _B_PER_W

    def one_table(ids_hbm, packed_hbm, out_hbm):
        stage = []
        for j in range(_NCHUNK):
            stage.append(pltpu.async_copy(
                ids_hbm.at[pl.ds(base + j * _CHUNK, _CHUNK)],
                idx_v.at[j], isem))
        for c in stage:
            c.wait()
        copies = []
        for j in range(_NCHUNK):
            copies.append(pltpu.async_copy(
                packed_hbm.at[idx_v.at[j]],
                rows4.at[pl.ds(j * _CHUNK, _CHUNK)], sem))
        for c in copies:
            c.wait()
        pltpu.sync_copy(rows4, out_hbm.at[pl.ds(base, _B_PER_W)])

    one_table(uid_hbm, up_hbm, out_u)
    one_table(iid_hbm, ip_hbm, out_i)


@functools.cache
def _sc_gather():
    return pl.kernel(
        _gather_body,
        out_type=(
            jax.ShapeDtypeStruct((BATCH, _PROW), jnp.float32),
            jax.ShapeDtypeStruct((BATCH, _PROW), jnp.float32),
        ),
        mesh=plsc.VectorSubcoreMesh(core_axis_name="c", subcore_axis_name="s"),
        scratch_types=[
            pltpu.VMEM((_NCHUNK, _CHUNK), jnp.int32),
            pltpu.VMEM((_B_PER_W, _PROW), jnp.float32),
            pltpu.SemaphoreType.DMA,
            pltpu.SemaphoreType.DMA,
        ],
    )


_TR_IN = 2048
_TR_GRID = 489
_TR_OUT = _TR_IN // _PACK
_NQ = _TR_GRID * _TR_OUT


def _pack_one(x):
    eye = jnp.eye(EMBED_DIM, dtype=jnp.float32)
    blks = []
    for tl in range(_TR_IN // 512):
        ta = x[:, 512 * tl:512 * tl + 256].T
        tb = lax.dot_general(x[:, 512 * tl + 256:512 * (tl + 1)], eye,
                             (((0,), (0,)), ((), ())),
                             preferred_element_type=jnp.float32)
        blks.append(jnp.concatenate(
            [ta[0:128], ta[128:256], tb[0:128], tb[128:256]], axis=1))
    return jnp.concatenate(blks, axis=0)


def _pack_body(u_ref, up_ref):
    i = pl.program_id(0)
    pu = _pack_one(u_ref[...])
    up_ref[...] = pu

    @pl.when(i == _TR_GRID - 1)
    def _():
        w = lax.broadcasted_iota(jnp.int32, (_TR_OUT, _PROW), 0)
        col = lax.broadcasted_iota(jnp.int32, (_TR_OUT, _PROW), 1)
        r_local = (512 * (w // 128) + 128 * (col // 32) + w % 128)
        valid = (i * _TR_IN + r_local) < 1000000
        up_ref[...] = jnp.where(valid, pu, 0.0)


def _pack_call(uet):
    return pl.pallas_call(
        _pack_body,
        grid=(_TR_GRID,),
        in_specs=[
            pl.BlockSpec((EMBED_DIM, _TR_IN), lambda i: (0, i)),
        ],
        out_specs=pl.BlockSpec((_TR_OUT, _PROW), lambda i: (i, 0)),
        out_shape=jax.ShapeDtypeStruct((_NQ, _PROW), jnp.float32),
    )(uet)


_SP_CHUNKS = 977
_SP_LAST = 999040
_NQS = _SP_CHUNKS * 256


def _spack_body(iet2_hbm, out_hbm, slab, obuf, sem):
    wid = lax.axis_index("s") * _NUM_CORES + lax.axis_index("c")
    lane = lax.iota(jnp.int32, 16)
    zero = lane * 0
    cxv = [lax.rem(lane + 16 * v, 32) for v in range(8)]

    def chunk_body(it, _):
        c = it * _NW + wid

        @pl.when(c < _SP_CHUNKS)
        def _():
            start = jnp.minimum(c * 1024, _SP_LAST)
            pltpu.async_copy(iet2_hbm.at[:, pl.ds(start, 1024)], slab,
                             sem).wait()

            def row_body(w, _2):
                for tl in range(2):
                    for v in range(8):
                        rl = zero + (w + 512 * tl + 128 * (v // 2))
                        vals = plsc.load_gather(slab, [cxv[v], rl])
                        obuf[128 * tl + w, pl.ds(16 * v, 16)] = vals
                return _2

            lax.fori_loop(0, 128, row_body, None)
            pltpu.sync_copy(obuf, out_hbm.at[pl.ds(c * 256, 256)])
        return _

    lax.fori_loop(0, (_SP_CHUNKS + _NW - 1) // _NW, chunk_body, None)


@functools.cache
def _sc_pack():
    return pl.kernel(
        _spack_body,
        out_type=jax.ShapeDtypeStruct((_NQS, _PROW), jnp.float32),
        mesh=plsc.VectorSubcoreMesh(core_axis_name="c", subcore_axis_name="s"),
        scratch_types=[
            pltpu.VMEM((EMBED_DIM, 1024), jnp.float32),
            pltpu.VMEM((256, _PROW), jnp.float32),
            pltpu.SemaphoreType.DMA,
        ],
        compiler_params=pltpu.CompilerParams(needs_layout_passes=False),
    )


_MLP_BLK = 2048


def _mlp_body(yu_ref, yi_ref, ohu_ref, ohi_ref, mu_ref, mi_ref, b1_ref,
              w2_ref, b2_ref, out_ref):
    h = jnp.zeros((_MLP_BLK, HIDDEN_DIM), dtype=jnp.float32)
    yu = yu_ref[...]
    yi = yi_ref[...]
    for a in range(_PACK):
        hu = jnp.dot(yu, mu_ref[a], preferred_element_type=jnp.float32)
        hi = jnp.dot(yi, mi_ref[a], preferred_element_type=jnp.float32)
        h = h + hu * ohu_ref[:, a:a + 1] + hi * ohi_ref[:, a:a + 1]
    h = jnp.maximum(h + b1_ref[...], 0.0)
    y = jnp.sum(h * w2_ref[...], axis=1) + b2_ref[0, 0]
    out_ref[...] = jax.nn.sigmoid(y)


def _mlp_call(yu, yi, ohu, ohi, mu, mi, b1, w2, b2):
    grid = BATCH // _MLP_BLK
    return pl.pallas_call(
        _mlp_body,
        grid=(grid,),
        in_specs=[
            pl.BlockSpec((_MLP_BLK, _PROW), lambda i: (i, 0)),
            pl.BlockSpec((_MLP_BLK, _PROW), lambda i: (i, 0)),
            pl.BlockSpec((_MLP_BLK, _PACK), lambda i: (i, 0)),
            pl.BlockSpec((_MLP_BLK, _PACK), lambda i: (i, 0)),
            pl.BlockSpec((_PACK, _PROW, HIDDEN_DIM), lambda i: (0, 0, 0)),
            pl.BlockSpec((_PACK, _PROW, HIDDEN_DIM), lambda i: (0, 0, 0)),
            pl.BlockSpec((1, HIDDEN_DIM), lambda i: (0, 0)),
            pl.BlockSpec((1, HIDDEN_DIM), lambda i: (0, 0)),
            pl.BlockSpec((1, 1), lambda i: (0, 0)),
        ],
        out_specs=pl.BlockSpec((_MLP_BLK,), lambda i: (i,)),
        out_shape=jax.ShapeDtypeStruct((BATCH,), jnp.float32),
    )(yu, yi, ohu, ohi, mu, mi, b1, w2, b2)


def _shifted_weights(w_half):
    mats = []
    for a in range(_PACK):
        m = jnp.zeros((_PROW, HIDDEN_DIM), dtype=jnp.float32)
        m = lax.dynamic_update_slice(m, w_half, (a * EMBED_DIM, 0))
        mats.append(m)
    return jnp.stack(mats)


def kernel(user_ids, item_ids, user_emb, item_emb, W1, b1, W2, b2):
    uet = user_emb.T
    up = _pack_call(uet)
    ipk = _sc_pack()(item_emb.T)
    qu = jnp.bitwise_or(jnp.left_shift(jnp.right_shift(user_ids, 9), 7),
                        jnp.bitwise_and(user_ids, 127))
    au = jnp.bitwise_and(jnp.right_shift(user_ids, 7), _PACK - 1)
    is_last = item_ids >= _SP_LAST
    ci = jnp.where(is_last, _SP_CHUNKS - 1, jnp.right_shift(item_ids, 10))
    rl = item_ids - jnp.where(is_last, _SP_LAST,
                              jnp.left_shift(jnp.right_shift(item_ids, 10),
                                             10))
    qi = (ci * 256
          + 128 * jnp.bitwise_and(jnp.right_shift(rl, 9), 1)
          + jnp.bitwise_and(rl, 127))
    ai = jnp.bitwise_and(jnp.right_shift(rl, 7), _PACK - 1)
    yu, yi = _sc_gather()(qu, qi, up, ipk)
    ohu = jax.nn.one_hot(au, _PACK, dtype=jnp.float32)
    ohi = jax.nn.one_hot(ai, _PACK, dtype=jnp.float32)
    mu = _shifted_weights(W1[:, :EMBED_DIM].T)
    mi = _shifted_weights(W1[:, EMBED_DIM:].T)
    b1r = b1.reshape(1, HIDDEN_DIM)
    w2r = W2.reshape(1, HIDDEN_DIM)
    b2r = b2.reshape(1, 1)
    return _mlp_call(yu, yi, ohu, ohi, mu, mi, b1r, w2r, b2r)


# speedup vs baseline: 1.4491x; 1.4491x over previous
import functools

import jax
import jax.numpy as jnp
from jax import lax
from jax.experimental import pallas as pl
from jax.experimental.pallas import tpu as pltpu
from jax.experimental.pallas import tpu_sc as plsc

BATCH = 16384
EMBED_DIM = 32
HIDDEN_DIM = 64
_PACK = 4
_PROW = _PACK * EMBED_DIM

_NUM_CORES = 2
_NUM_SUBCORES = 16
_NW = _NUM_CORES * _NUM_SUBCORES
_B_PER_W = BATCH // _NW
_CHUNK = 128
_NCHUNK = _B_PER_W // _CHUNK


def _gather_body(uid_hbm, iid_hbm, up_hbm, ip_hbm, out_u, out_i,
                 idx_v, rows4, sem, isem):
    wid = lax.axis_index("s") * _NUM_CORES + lax.axis_index("c")
    base = wid * _B_PER_W

    def one_table(ids_hbm, packed_hbm, out_hbm):
        stage = []
        for j in range(_NCHUNK):
            stage.append(pltpu.async_copy(
                ids_hbm.at[pl.ds(base + j * _CHUNK, _CHUNK)],
                idx_v.at[j], isem))
        for c in stage:
            c.wait()
        copies = []
        for j in range(_NCHUNK):
            copies.append(pltpu.async_copy(
                packed_hbm.at[idx_v.at[j]],
                rows4.at[pl.ds(j * _CHUNK, _CHUNK)], sem))
        for c in copies:
            c.wait()
        pltpu.sync_copy(rows4, out_hbm.at[pl.ds(base, _B_PER_W)])

    one_table(uid_hbm, up_hbm, out_u)
    one_table(iid_hbm, ip_hbm, out_i)


@functools.cache
def _sc_gather():
    return pl.kernel(
        _gather_body,
        out_type=(
            jax.ShapeDtypeStruct((BATCH, _PROW), jnp.float32),
            jax.ShapeDtypeStruct((BATCH, _PROW), jnp.float32),
        ),
        mesh=plsc.VectorSubcoreMesh(core_axis_name="c", subcore_axis_name="s"),
        scratch_types=[
            pltpu.VMEM((_NCHUNK, _CHUNK), jnp.int32),
            pltpu.VMEM((_B_PER_W, _PROW), jnp.float32),
            pltpu.SemaphoreType.DMA,
            pltpu.SemaphoreType.DMA,
        ],
    )


_TR_IN = 2048
_TR_GRID = 489
_TR_OUT = _TR_IN // _PACK
_NQ = _TR_GRID * _TR_OUT


def _pack_one(x):
    eye = jnp.eye(EMBED_DIM, dtype=jnp.float32)
    blks = []
    for tl in range(_TR_IN // 512):
        ta = x[:, 512 * tl:512 * tl + 256].T
        tb = lax.dot_general(x[:, 512 * tl + 256:512 * (tl + 1)], eye,
                             (((0,), (0,)), ((), ())),
                             preferred_element_type=jnp.float32)
        blks.append(jnp.concatenate(
            [ta[0:128], ta[128:256], tb[0:128], tb[128:256]], axis=1))
    return jnp.concatenate(blks, axis=0)


def _pack_body(u_ref, i_ref, up_ref, ip_ref):
    i = pl.program_id(0)
    pu = _pack_one(u_ref[...])
    pi = _pack_one(i_ref[...])
    up_ref[...] = pu
    ip_ref[...] = pi

    @pl.when(i == _TR_GRID - 1)
    def _():
        w = lax.broadcasted_iota(jnp.int32, (_TR_OUT, _PROW), 0)
        col = lax.broadcasted_iota(jnp.int32, (_TR_OUT, _PROW), 1)
        r_local = (512 * (w // 128) + 128 * (col // 32) + w % 128)
        valid = (i * _TR_IN + r_local) < 1000000
        up_ref[...] = jnp.where(valid, pu, 0.0)
        ip_ref[...] = jnp.where(valid, pi, 0.0)


def _pack_call(uet, iet):
    return pl.pallas_call(
        _pack_body,
        grid=(_TR_GRID,),
        in_specs=[
            pl.BlockSpec((EMBED_DIM, _TR_IN), lambda i: (0, i)),
            pl.BlockSpec((EMBED_DIM, _TR_IN), lambda i: (0, i)),
        ],
        out_specs=(pl.BlockSpec((_TR_OUT, _PROW), lambda i: (i, 0)),
                   pl.BlockSpec((_TR_OUT, _PROW), lambda i: (i, 0))),
        out_shape=(jax.ShapeDtypeStruct((_NQ, _PROW), jnp.float32),
                   jax.ShapeDtypeStruct((_NQ, _PROW), jnp.float32)),
    )(uet, iet)


_MLP_BLK = 2048


def _mlp_body(yu_ref, yi_ref, ohu_ref, ohi_ref, mu_ref, mi_ref, b1_ref,
              w2_ref, b2_ref, out_ref):
    h = jnp.zeros((_MLP_BLK, HIDDEN_DIM), dtype=jnp.float32)
    yu = yu_ref[...]
    yi = yi_ref[...]
    for a in range(_PACK):
        hu = jnp.dot(yu, mu_ref[a], preferred_element_type=jnp.float32)
        hi = jnp.dot(yi, mi_ref[a], preferred_element_type=jnp.float32)
        h = h + hu * ohu_ref[:, a:a + 1] + hi * ohi_ref[:, a:a + 1]
    h = jnp.maximum(h + b1_ref[...], 0.0)
    y = jnp.sum(h * w2_ref[...], axis=1) + b2_ref[0, 0]
    out_ref[...] = jax.nn.sigmoid(y)


def _mlp_call(yu, yi, ohu, ohi, mu, mi, b1, w2, b2):
    grid = BATCH // _MLP_BLK
    return pl.pallas_call(
        _mlp_body,
        grid=(grid,),
        in_specs=[
            pl.BlockSpec((_MLP_BLK, _PROW), lambda i: (i, 0)),
            pl.BlockSpec((_MLP_BLK, _PROW), lambda i: (i, 0)),
            pl.BlockSpec((_MLP_BLK, _PACK), lambda i: (i, 0)),
            pl.BlockSpec((_MLP_BLK, _PACK), lambda i: (i, 0)),
            pl.BlockSpec((_PACK, _PROW, HIDDEN_DIM), lambda i: (0, 0, 0)),
            pl.BlockSpec((_PACK, _PROW, HIDDEN_DIM), lambda i: (0, 0, 0)),
            pl.BlockSpec((1, HIDDEN_DIM), lambda i: (0, 0)),
            pl.BlockSpec((1, HIDDEN_DIM), lambda i: (0, 0)),
            pl.BlockSpec((1, 1), lambda i: (0, 0)),
        ],
        out_specs=pl.BlockSpec((_MLP_BLK,), lambda i: (i,)),
        out_shape=jax.ShapeDtypeStruct((BATCH,), jnp.float32),
    )(yu, yi, ohu, ohi, mu, mi, b1, w2, b2)


def _shifted_weights(w_half):
    mats = []
    for a in range(_PACK):
        m = jnp.zeros((_PROW, HIDDEN_DIM), dtype=jnp.float32)
        m = lax.dynamic_update_slice(m, w_half, (a * EMBED_DIM, 0))
        mats.append(m)
    return jnp.stack(mats)


def kernel(user_ids, item_ids, user_emb, item_emb, W1, b1, W2, b2):
    uet = user_emb.T
    iet = item_emb.T
    up, ip = _pack_call(uet, iet)
    qu = jnp.bitwise_or(jnp.left_shift(jnp.right_shift(user_ids, 9), 7),
                        jnp.bitwise_and(user_ids, 127))
    qi = jnp.bitwise_or(jnp.left_shift(jnp.right_shift(item_ids, 9), 7),
                        jnp.bitwise_and(item_ids, 127))
    yu, yi = _sc_gather()(qu, qi, up, ip)
    au = jnp.bitwise_and(jnp.right_shift(user_ids, 7), _PACK - 1)
    ai = jnp.bitwise_and(jnp.right_shift(item_ids, 7), _PACK - 1)
    ohu = jax.nn.one_hot(au, _PACK, dtype=jnp.float32)
    ohi = jax.nn.one_hot(ai, _PACK, dtype=jnp.float32)
    mu = _shifted_weights(W1[:, :EMBED_DIM].T)
    mi = _shifted_weights(W1[:, EMBED_DIM:].T)
    b1r = b1.reshape(1, HIDDEN_DIM)
    w2r = W2.reshape(1, HIDDEN_DIM)
    b2r = b2.reshape(1, 1)
    return _mlp_call(yu, yi, ohu, ohi, mu, mi, b1r, w2r, b2r)


# speedup vs baseline: 1.6657x; 1.1495x over previous
import functools

import jax
import jax.numpy as jnp
from jax import lax
from jax.experimental import pallas as pl
from jax.experimental.pallas import tpu as pltpu
from jax.experimental.pallas import tpu_sc as plsc

BATCH = 16384
EMBED_DIM = 32
HIDDEN_DIM = 64
_PACK = 4
_PROW = _PACK * EMBED_DIM

_NUM_CORES = 2
_NUM_SUBCORES = 16
_NW = _NUM_CORES * _NUM_SUBCORES
_B_PER_W = BATCH // _NW
_CHUNK = 128
_NCHUNK = _B_PER_W // _CHUNK


def _gather_body(uid_hbm, iid_hbm, up_hbm, ip_hbm, out_u, out_i,
                 idx_v, rows4, sem, isem):
    wid = lax.axis_index("s") * _NUM_CORES + lax.axis_index("c")
    base = wid * _B_PER_W

    def one_table(ids_hbm, packed_hbm, out_hbm):
        stage = []
        for j in range(_NCHUNK):
            stage.append(pltpu.async_copy(
                ids_hbm.at[pl.ds(base + j * _CHUNK, _CHUNK)],
                idx_v.at[j], isem))
        for c in stage:
            c.wait()
        copies = []
        for j in range(_NCHUNK):
            copies.append(pltpu.async_copy(
                packed_hbm.at[idx_v.at[j]],
                rows4.at[pl.ds(j * _CHUNK, _CHUNK)], sem))
        for c in copies:
            c.wait()
        pltpu.sync_copy(rows4, out_hbm.at[pl.ds(base, _B_PER_W)])

    one_table(uid_hbm, up_hbm, out_u)
    one_table(iid_hbm, ip_hbm, out_i)


@functools.cache
def _sc_gather():
    return pl.kernel(
        _gather_body,
        out_type=(
            jax.ShapeDtypeStruct((BATCH, _PROW), jnp.float32),
            jax.ShapeDtypeStruct((BATCH, _PROW), jnp.float32),
        ),
        mesh=plsc.VectorSubcoreMesh(core_axis_name="c", subcore_axis_name="s"),
        scratch_types=[
            pltpu.VMEM((_NCHUNK, _CHUNK), jnp.int32),
            pltpu.VMEM((_B_PER_W, _PROW), jnp.float32),
            pltpu.SemaphoreType.DMA,
            pltpu.SemaphoreType.DMA,
        ],
    )


_TR_IN = 4096
_TR_GRID = 245
_TR_OUT = _TR_IN // _PACK
_NQ = _TR_GRID * _TR_OUT


def _pack_one(x):
    eye = jnp.eye(EMBED_DIM, dtype=jnp.float32)
    blks = []
    for tl in range(_TR_IN // 512):
        ta = x[:, 512 * tl:512 * tl + 256].T
        tb = lax.dot_general(x[:, 512 * tl + 256:512 * (tl + 1)], eye,
                             (((0,), (0,)), ((), ())),
                             preferred_element_type=jnp.float32)
        blks.append(jnp.concatenate(
            [ta[0:128], ta[128:256], tb[0:128], tb[128:256]], axis=1))
    return jnp.concatenate(blks, axis=0)


def _pack_body(u_ref, i_ref, up_ref, ip_ref):
    i = pl.program_id(0)
    pu = _pack_one(u_ref[...])
    pi = _pack_one(i_ref[...])
    up_ref[...] = pu
    ip_ref[...] = pi

    @pl.when(i == _TR_GRID - 1)
    def _():
        w = lax.broadcasted_iota(jnp.int32, (_TR_OUT, _PROW), 0)
        col = lax.broadcasted_iota(jnp.int32, (_TR_OUT, _PROW), 1)
        r_local = (512 * (w // 128) + 128 * (col // 32) + w % 128)
        valid = (i * _TR_IN + r_local) < 1000000
        up_ref[...] = jnp.where(valid, pu, 0.0)
        ip_ref[...] = jnp.where(valid, pi, 0.0)


def _pack_call(uet, iet):
    return pl.pallas_call(
        _pack_body,
        grid=(_TR_GRID,),
        in_specs=[
            pl.BlockSpec((EMBED_DIM, _TR_IN), lambda i: (0, i)),
            pl.BlockSpec((EMBED_DIM, _TR_IN), lambda i: (0, i)),
        ],
        out_specs=(pl.BlockSpec((_TR_OUT, _PROW), lambda i: (i, 0)),
                   pl.BlockSpec((_TR_OUT, _PROW), lambda i: (i, 0))),
        out_shape=(jax.ShapeDtypeStruct((_NQ, _PROW), jnp.float32),
                   jax.ShapeDtypeStruct((_NQ, _PROW), jnp.float32)),
    )(uet, iet)


_MLP_BLK = 2048


def _mlp_body(yu_ref, yi_ref, ohu_ref, ohi_ref, mu_ref, mi_ref, b1_ref,
              w2_ref, b2_ref, out_ref):
    h = jnp.zeros((_MLP_BLK, HIDDEN_DIM), dtype=jnp.float32)
    yu = yu_ref[...]
    yi = yi_ref[...]
    for a in range(_PACK):
        hu = jnp.dot(yu, mu_ref[a], preferred_element_type=jnp.float32)
        hi = jnp.dot(yi, mi_ref[a], preferred_element_type=jnp.float32)
        h = h + hu * ohu_ref[:, a:a + 1] + hi * ohi_ref[:, a:a + 1]
    h = jnp.maximum(h + b1_ref[...], 0.0)
    y = jnp.sum(h * w2_ref[...], axis=1) + b2_ref[0, 0]
    out_ref[...] = jax.nn.sigmoid(y)


def _mlp_call(yu, yi, ohu, ohi, mu, mi, b1, w2, b2):
    grid = BATCH // _MLP_BLK
    return pl.pallas_call(
        _mlp_body,
        grid=(grid,),
        in_specs=[
            pl.BlockSpec((_MLP_BLK, _PROW), lambda i: (i, 0)),
            pl.BlockSpec((_MLP_BLK, _PROW), lambda i: (i, 0)),
            pl.BlockSpec((_MLP_BLK, _PACK), lambda i: (i, 0)),
            pl.BlockSpec((_MLP_BLK, _PACK), lambda i: (i, 0)),
            pl.BlockSpec((_PACK, _PROW, HIDDEN_DIM), lambda i: (0, 0, 0)),
            pl.BlockSpec((_PACK, _PROW, HIDDEN_DIM), lambda i: (0, 0, 0)),
            pl.BlockSpec((1, HIDDEN_DIM), lambda i: (0, 0)),
            pl.BlockSpec((1, HIDDEN_DIM), lambda i: (0, 0)),
            pl.BlockSpec((1, 1), lambda i: (0, 0)),
        ],
        out_specs=pl.BlockSpec((_MLP_BLK,), lambda i: (i,)),
        out_shape=jax.ShapeDtypeStruct((BATCH,), jnp.float32),
    )(yu, yi, ohu, ohi, mu, mi, b1, w2, b2)


def _shifted_weights(w_half):
    mats = []
    for a in range(_PACK):
        m = jnp.zeros((_PROW, HIDDEN_DIM), dtype=jnp.float32)
        m = lax.dynamic_update_slice(m, w_half, (a * EMBED_DIM, 0))
        mats.append(m)
    return jnp.stack(mats)


def kernel(user_ids, item_ids, user_emb, item_emb, W1, b1, W2, b2):
    uet = user_emb.T
    iet = item_emb.T
    up, ip = _pack_call(uet, iet)
    qu = jnp.bitwise_or(jnp.left_shift(jnp.right_shift(user_ids, 9), 7),
                        jnp.bitwise_and(user_ids, 127))
    qi = jnp.bitwise_or(jnp.left_shift(jnp.right_shift(item_ids, 9), 7),
                        jnp.bitwise_and(item_ids, 127))
    yu, yi = _sc_gather()(qu, qi, up, ip)
    au = jnp.bitwise_and(jnp.right_shift(user_ids, 7), _PACK - 1)
    ai = jnp.bitwise_and(jnp.right_shift(item_ids, 7), _PACK - 1)
    ohu = jax.nn.one_hot(au, _PACK, dtype=jnp.float32)
    ohi = jax.nn.one_hot(ai, _PACK, dtype=jnp.float32)
    mu = _shifted_weights(W1[:, :EMBED_DIM].T)
    mi = _shifted_weights(W1[:, EMBED_DIM:].T)
    b1r = b1.reshape(1, HIDDEN_DIM)
    w2r = W2.reshape(1, HIDDEN_DIM)
    b2r = b2.reshape(1, 1)
    return _mlp_call(yu, yi, ohu, ohi, mu, mi, b1r, w2r, b2r)


# speedup vs baseline: 1.7083x; 1.0256x over previous
import functools

import jax
import jax.numpy as jnp
from jax import lax
from jax.experimental import pallas as pl
from jax.experimental.pallas import tpu as pltpu
from jax.experimental.pallas import tpu_sc as plsc

BATCH = 16384
EMBED_DIM = 32
HIDDEN_DIM = 64
_PACK = 4
_PROW = _PACK * EMBED_DIM

_NUM_CORES = 2
_NUM_SUBCORES = 16
_NW = _NUM_CORES * _NUM_SUBCORES
_B_PER_W = BATCH // _NW
_CHUNK = 128
_NCHUNK = _B_PER_W // _CHUNK


def _gather_body(uid_hbm, iid_hbm, up_hbm, ip_hbm, out_u, out_i,
                 idx_v, rows4, sem, isem):
    wid = lax.axis_index("s") * _NUM_CORES + lax.axis_index("c")
    base = wid * _B_PER_W

    def one_table(ids_hbm, packed_hbm, out_hbm):
        stage = []
        for j in range(_NCHUNK):
            stage.append(pltpu.async_copy(
                ids_hbm.at[pl.ds(base + j * _CHUNK, _CHUNK)],
                idx_v.at[j], isem))
        for c in stage:
            c.wait()
        copies = []
        for j in range(_NCHUNK):
            copies.append(pltpu.async_copy(
                packed_hbm.at[idx_v.at[j]],
                rows4.at[pl.ds(j * _CHUNK, _CHUNK)], sem))
        for c in copies:
            c.wait()
        pltpu.sync_copy(rows4, out_hbm.at[pl.ds(base, _B_PER_W)])

    one_table(uid_hbm, up_hbm, out_u)
    one_table(iid_hbm, ip_hbm, out_i)


@functools.cache
def _sc_gather():
    return pl.kernel(
        _gather_body,
        out_type=(
            jax.ShapeDtypeStruct((BATCH, _PROW), jnp.float32),
            jax.ShapeDtypeStruct((BATCH, _PROW), jnp.float32),
        ),
        mesh=plsc.VectorSubcoreMesh(core_axis_name="c", subcore_axis_name="s"),
        scratch_types=[
            pltpu.VMEM((_NCHUNK, _CHUNK), jnp.int32),
            pltpu.VMEM((_B_PER_W, _PROW), jnp.float32),
            pltpu.SemaphoreType.DMA,
            pltpu.SemaphoreType.DMA,
        ],
    )


_TR_IN = 8192
_TR_GRID = 123
_TR_OUT = _TR_IN // _PACK
_NQ = _TR_GRID * _TR_OUT


def _pack_one(x):
    eye = jnp.eye(EMBED_DIM, dtype=jnp.float32)
    blks = []
    for tl in range(_TR_IN // 512):
        ta = x[:, 512 * tl:512 * tl + 256].T
        tb = lax.dot_general(x[:, 512 * tl + 256:512 * (tl + 1)], eye,
                             (((0,), (0,)), ((), ())),
                             preferred_element_type=jnp.float32)
        blks.append(jnp.concatenate(
            [ta[0:128], ta[128:256], tb[0:128], tb[128:256]], axis=1))
    return jnp.concatenate(blks, axis=0)


def _pack_body(u_ref, i_ref, up_ref, ip_ref):
    i = pl.program_id(0)
    pu = _pack_one(u_ref[...])
    pi = _pack_one(i_ref[...])
    up_ref[...] = pu
    ip_ref[...] = pi

    @pl.when(i == _TR_GRID - 1)
    def _():
        w = lax.broadcasted_iota(jnp.int32, (_TR_OUT, _PROW), 0)
        col = lax.broadcasted_iota(jnp.int32, (_TR_OUT, _PROW), 1)
        r_local = (512 * (w // 128) + 128 * (col // 32) + w % 128)
        valid = (i * _TR_IN + r_local) < 1000000
        up_ref[...] = jnp.where(valid, pu, 0.0)
        ip_ref[...] = jnp.where(valid, pi, 0.0)


def _pack_call(uet, iet):
    return pl.pallas_call(
        _pack_body,
        grid=(_TR_GRID,),
        in_specs=[
            pl.BlockSpec((EMBED_DIM, _TR_IN), lambda i: (0, i)),
            pl.BlockSpec((EMBED_DIM, _TR_IN), lambda i: (0, i)),
        ],
        out_specs=(pl.BlockSpec((_TR_OUT, _PROW), lambda i: (i, 0)),
                   pl.BlockSpec((_TR_OUT, _PROW), lambda i: (i, 0))),
        out_shape=(jax.ShapeDtypeStruct((_NQ, _PROW), jnp.float32),
                   jax.ShapeDtypeStruct((_NQ, _PROW), jnp.float32)),
    )(uet, iet)


_MLP_BLK = 2048


def _mlp_body(yu_ref, yi_ref, ohu_ref, ohi_ref, mu_ref, mi_ref, b1_ref,
              w2_ref, b2_ref, out_ref):
    h = jnp.zeros((_MLP_BLK, HIDDEN_DIM), dtype=jnp.float32)
    yu = yu_ref[...]
    yi = yi_ref[...]
    for a in range(_PACK):
        hu = jnp.dot(yu, mu_ref[a], preferred_element_type=jnp.float32)
        hi = jnp.dot(yi, mi_ref[a], preferred_element_type=jnp.float32)
        h = h + hu * ohu_ref[:, a:a + 1] + hi * ohi_ref[:, a:a + 1]
    h = jnp.maximum(h + b1_ref[...], 0.0)
    y = jnp.sum(h * w2_ref[...], axis=1) + b2_ref[0, 0]
    out_ref[...] = jax.nn.sigmoid(y)


def _mlp_call(yu, yi, ohu, ohi, mu, mi, b1, w2, b2):
    grid = BATCH // _MLP_BLK
    return pl.pallas_call(
        _mlp_body,
        grid=(grid,),
        in_specs=[
            pl.BlockSpec((_MLP_BLK, _PROW), lambda i: (i, 0)),
            pl.BlockSpec((_MLP_BLK, _PROW), lambda i: (i, 0)),
            pl.BlockSpec((_MLP_BLK, _PACK), lambda i: (i, 0)),
            pl.BlockSpec((_MLP_BLK, _PACK), lambda i: (i, 0)),
            pl.BlockSpec((_PACK, _PROW, HIDDEN_DIM), lambda i: (0, 0, 0)),
            pl.BlockSpec((_PACK, _PROW, HIDDEN_DIM), lambda i: (0, 0, 0)),
            pl.BlockSpec((1, HIDDEN_DIM), lambda i: (0, 0)),
            pl.BlockSpec((1, HIDDEN_DIM), lambda i: (0, 0)),
            pl.BlockSpec((1, 1), lambda i: (0, 0)),
        ],
        out_specs=pl.BlockSpec((_MLP_BLK,), lambda i: (i,)),
        out_shape=jax.ShapeDtypeStruct((BATCH,), jnp.float32),
    )(yu, yi, ohu, ohi, mu, mi, b1, w2, b2)


def _shifted_weights(w_half):
    mats = []
    for a in range(_PACK):
        m = jnp.zeros((_PROW, HIDDEN_DIM), dtype=jnp.float32)
        m = lax.dynamic_update_slice(m, w_half, (a * EMBED_DIM, 0))
        mats.append(m)
    return jnp.stack(mats)


def kernel(user_ids, item_ids, user_emb, item_emb, W1, b1, W2, b2):
    uet = user_emb.T
    iet = item_emb.T
    up, ip = _pack_call(uet, iet)
    qu = jnp.bitwise_or(jnp.left_shift(jnp.right_shift(user_ids, 9), 7),
                        jnp.bitwise_and(user_ids, 127))
    qi = jnp.bitwise_or(jnp.left_shift(jnp.right_shift(item_ids, 9), 7),
                        jnp.bitwise_and(item_ids, 127))
    yu, yi = _sc_gather()(qu, qi, up, ip)
    au = jnp.bitwise_and(jnp.right_shift(user_ids, 7), _PACK - 1)
    ai = jnp.bitwise_and(jnp.right_shift(item_ids, 7), _PACK - 1)
    ohu = jax.nn.one_hot(au, _PACK, dtype=jnp.float32)
    ohi = jax.nn.one_hot(ai, _PACK, dtype=jnp.float32)
    mu = _shifted_weights(W1[:, :EMBED_DIM].T)
    mi = _shifted_weights(W1[:, EMBED_DIM:].T)
    b1r = b1.reshape(1, HIDDEN_DIM)
    w2r = W2.reshape(1, HIDDEN_DIM)
    b2r = b2.reshape(1, 1)
    return _mlp_call(yu, yi, ohu, ohi, mu, mi, b1r, w2r, b2r)


# speedup vs baseline: 1.7112x; 1.0017x over previous
import functools

import jax
import jax.numpy as jnp
from jax import lax
from jax.experimental import pallas as pl
from jax.experimental.pallas import tpu as pltpu
from jax.experimental.pallas import tpu_sc as plsc

BATCH = 16384
EMBED_DIM = 32
HIDDEN_DIM = 64
_PACK = 4
_PROW = _PACK * EMBED_DIM

_NUM_CORES = 2
_NUM_SUBCORES = 16
_NW = _NUM_CORES * _NUM_SUBCORES
_B_PER_W = BATCH // _NW
_CHUNK = 128
_NCHUNK = _B_PER_W // _CHUNK


def _gather_body(uid_hbm, iid_hbm, up_hbm, ip_hbm, out_u, out_i,
                 idx_v, rows4, sem, isem):
    wid = lax.axis_index("s") * _NUM_CORES + lax.axis_index("c")
    base = wid * _B_PER_W

    def one_table(ids_hbm, packed_hbm, out_hbm):
        stage = []
        for j in range(_NCHUNK):
            stage.append(pltpu.async_copy(
                ids_hbm.at[pl.ds(base + j * _CHUNK, _CHUNK)],
                idx_v.at[j], isem))
        for c in stage:
            c.wait()
        copies = []
        for j in range(_NCHUNK):
            copies.append(pltpu.async_copy(
                packed_hbm.at[idx_v.at[j]],
                rows4.at[pl.ds(j * _CHUNK, _CHUNK)], sem))
        for c in copies:
            c.wait()
        pltpu.sync_copy(rows4, out_hbm.at[pl.ds(base, _B_PER_W)])

    one_table(uid_hbm, up_hbm, out_u)
    one_table(iid_hbm, ip_hbm, out_i)


@functools.cache
def _sc_gather():
    return pl.kernel(
        _gather_body,
        out_type=(
            jax.ShapeDtypeStruct((BATCH, _PROW), jnp.float32),
            jax.ShapeDtypeStruct((BATCH, _PROW), jnp.float32),
        ),
        mesh=plsc.VectorSubcoreMesh(core_axis_name="c", subcore_axis_name="s"),
        scratch_types=[
            pltpu.VMEM((_NCHUNK, _CHUNK), jnp.int32),
            pltpu.VMEM((_B_PER_W, _PROW), jnp.float32),
            pltpu.SemaphoreType.DMA,
            pltpu.SemaphoreType.DMA,
        ],
    )


_TR_IN = 16384
_TR_GRID = 62
_TR_OUT = _TR_IN // _PACK
_NQ = _TR_GRID * _TR_OUT


def _pack_one(x):
    eye = jnp.eye(EMBED_DIM, dtype=jnp.float32)
    blks = []
    for tl in range(_TR_IN // 512):
        ta = x[:, 512 * tl:512 * tl + 256].T
        tb = lax.dot_general(x[:, 512 * tl + 256:512 * (tl + 1)], eye,
                             (((0,), (0,)), ((), ())),
                             preferred_element_type=jnp.float32)
        blks.append(jnp.concatenate(
            [ta[0:128], ta[128:256], tb[0:128], tb[128:256]], axis=1))
    return jnp.concatenate(blks, axis=0)


def _pack_body(u_ref, i_ref, up_ref, ip_ref):
    i = pl.program_id(0)
    pu = _pack_one(u_ref[...])
    pi = _pack_one(i_ref[...])
    up_ref[...] = pu
    ip_ref[...] = pi

    @pl.when(i == _TR_GRID - 1)
    def _():
        w = lax.broadcasted_iota(jnp.int32, (_TR_OUT, _PROW), 0)
        col = lax.broadcasted_iota(jnp.int32, (_TR_OUT, _PROW), 1)
        r_local = (512 * (w // 128) + 128 * (col // 32) + w % 128)
        valid = (i * _TR_IN + r_local) < 1000000
        up_ref[...] = jnp.where(valid, pu, 0.0)
        ip_ref[...] = jnp.where(valid, pi, 0.0)


def _pack_call(uet, iet):
    return pl.pallas_call(
        _pack_body,
        grid=(_TR_GRID,),
        in_specs=[
            pl.BlockSpec((EMBED_DIM, _TR_IN), lambda i: (0, i)),
            pl.BlockSpec((EMBED_DIM, _TR_IN), lambda i: (0, i)),
        ],
        out_specs=(pl.BlockSpec((_TR_OUT, _PROW), lambda i: (i, 0)),
                   pl.BlockSpec((_TR_OUT, _PROW), lambda i: (i, 0))),
        out_shape=(jax.ShapeDtypeStruct((_NQ, _PROW), jnp.float32),
                   jax.ShapeDtypeStruct((_NQ, _PROW), jnp.float32)),
    )(uet, iet)


_MLP_BLK = 2048


def _mlp_body(yu_ref, yi_ref, ohu_ref, ohi_ref, mu_ref, mi_ref, b1_ref,
              w2_ref, b2_ref, out_ref):
    h = jnp.zeros((_MLP_BLK, HIDDEN_DIM), dtype=jnp.float32)
    yu = yu_ref[...]
    yi = yi_ref[...]
    for a in range(_PACK):
        hu = jnp.dot(yu, mu_ref[a], preferred_element_type=jnp.float32)
        hi = jnp.dot(yi, mi_ref[a], preferred_element_type=jnp.float32)
        h = h + hu * ohu_ref[:, a:a + 1] + hi * ohi_ref[:, a:a + 1]
    h = jnp.maximum(h + b1_ref[...], 0.0)
    y = jnp.sum(h * w2_ref[...], axis=1) + b2_ref[0, 0]
    out_ref[...] = jax.nn.sigmoid(y)


def _mlp_call(yu, yi, ohu, ohi, mu, mi, b1, w2, b2):
    grid = BATCH // _MLP_BLK
    return pl.pallas_call(
        _mlp_body,
        grid=(grid,),
        in_specs=[
            pl.BlockSpec((_MLP_BLK, _PROW), lambda i: (i, 0)),
            pl.BlockSpec((_MLP_BLK, _PROW), lambda i: (i, 0)),
            pl.BlockSpec((_MLP_BLK, _PACK), lambda i: (i, 0)),
            pl.BlockSpec((_MLP_BLK, _PACK), lambda i: (i, 0)),
            pl.BlockSpec((_PACK, _PROW, HIDDEN_DIM), lambda i: (0, 0, 0)),
            pl.BlockSpec((_PACK, _PROW, HIDDEN_DIM), lambda i: (0, 0, 0)),
            pl.BlockSpec((1, HIDDEN_DIM), lambda i: (0, 0)),
            pl.BlockSpec((1, HIDDEN_DIM), lambda i: (0, 0)),
            pl.BlockSpec((1, 1), lambda i: (0, 0)),
        ],
        out_specs=pl.BlockSpec((_MLP_BLK,), lambda i: (i,)),
        out_shape=jax.ShapeDtypeStruct((BATCH,), jnp.float32),
    )(yu, yi, ohu, ohi, mu, mi, b1, w2, b2)


def _shifted_weights(w_half):
    mats = []
    for a in range(_PACK):
        m = jnp.zeros((_PROW, HIDDEN_DIM), dtype=jnp.float32)
        m = lax.dynamic_update_slice(m, w_half, (a * EMBED_DIM, 0))
        mats.append(m)
    return jnp.stack(mats)


def kernel(user_ids, item_ids, user_emb, item_emb, W1, b1, W2, b2):
    uet = user_emb.T
    iet = item_emb.T
    up, ip = _pack_call(uet, iet)
    qu = jnp.bitwise_or(jnp.left_shift(jnp.right_shift(user_ids, 9), 7),
                        jnp.bitwise_and(user_ids, 127))
    qi = jnp.bitwise_or(jnp.left_shift(jnp.right_shift(item_ids, 9), 7),
                        jnp.bitwise_and(item_ids, 127))
    yu, yi = _sc_gather()(qu, qi, up, ip)
    au = jnp.bitwise_and(jnp.right_shift(user_ids, 7), _PACK - 1)
    ai = jnp.bitwise_and(jnp.right_shift(item_ids, 7), _PACK - 1)
    ohu = jax.nn.one_hot(au, _PACK, dtype=jnp.float32)
    ohi = jax.nn.one_hot(ai, _PACK, dtype=jnp.float32)
    mu = _shifted_weights(W1[:, :EMBED_DIM].T)
    mi = _shifted_weights(W1[:, EMBED_DIM:].T)
    b1r = b1.reshape(1, HIDDEN_DIM)
    w2r = W2.reshape(1, HIDDEN_DIM)
    b2r = b2.reshape(1, 1)
    return _mlp_call(yu, yi, ohu, ohi, mu, mi, b1r, w2r, b2r)


# speedup vs baseline: 1.7189x; 1.0045x over previous
import functools

import jax
import jax.numpy as jnp
from jax import lax
from jax.experimental import pallas as pl
from jax.experimental.pallas import tpu as pltpu
from jax.experimental.pallas import tpu_sc as plsc

BATCH = 16384
EMBED_DIM = 32
HIDDEN_DIM = 64
_PACK = 4
_PROW = _PACK * EMBED_DIM

_NUM_CORES = 2
_NUM_SUBCORES = 16
_NW = _NUM_CORES * _NUM_SUBCORES
_B_PER_W = BATCH // _NW
_CHUNK = 128
_NCHUNK = _B_PER_W // _CHUNK


def _gather_body(uid_hbm, iid_hbm, up_hbm, ip_hbm, out_u, out_i,
                 idx_v, rows4, sem, isem):
    wid = lax.axis_index("s") * _NUM_CORES + lax.axis_index("c")
    base = wid * _B_PER_W

    def one_table(ids_hbm, packed_hbm, out_hbm):
        stage = []
        for j in range(_NCHUNK):
            stage.append(pltpu.async_copy(
                ids_hbm.at[pl.ds(base + j * _CHUNK, _CHUNK)],
                idx_v.at[j], isem))
        for c in stage:
            c.wait()
        copies = []
        for j in range(_NCHUNK):
            copies.append(pltpu.async_copy(
                packed_hbm.at[idx_v.at[j]],
                rows4.at[pl.ds(j * _CHUNK, _CHUNK)], sem))
        for c in copies:
            c.wait()
        pltpu.sync_copy(rows4, out_hbm.at[pl.ds(base, _B_PER_W)])

    one_table(uid_hbm, up_hbm, out_u)
    one_table(iid_hbm, ip_hbm, out_i)


@functools.cache
def _sc_gather():
    return pl.kernel(
        _gather_body,
        out_type=(
            jax.ShapeDtypeStruct((BATCH, _PROW), jnp.float32),
            jax.ShapeDtypeStruct((BATCH, _PROW), jnp.float32),
        ),
        mesh=plsc.VectorSubcoreMesh(core_axis_name="c", subcore_axis_name="s"),
        scratch_types=[
            pltpu.VMEM((_NCHUNK, _CHUNK), jnp.int32),
            pltpu.VMEM((_B_PER_W, _PROW), jnp.float32),
            pltpu.SemaphoreType.DMA,
            pltpu.SemaphoreType.DMA,
        ],
    )


_TR_IN = 16384
_TR_GRID = 62
_TR_OUT = _TR_IN // _PACK
_NQ = _TR_GRID * _TR_OUT


def _pack_one(x):
    blks = []
    for tl in range(_TR_IN // 512):
        t = x[:, 512 * tl:512 * (tl + 1)].T
        blks.append(jnp.concatenate(
            [t[0:128], t[128:256], t[256:384], t[384:512]], axis=1))
    return jnp.concatenate(blks, axis=0)


def _pack_body(u_ref, i_ref, up_ref, ip_ref):
    i = pl.program_id(0)
    pu = _pack_one(u_ref[...])
    pi = _pack_one(i_ref[...])
    up_ref[...] = pu
    ip_ref[...] = pi

    @pl.when(i == _TR_GRID - 1)
    def _():
        w = lax.broadcasted_iota(jnp.int32, (_TR_OUT, _PROW), 0)
        col = lax.broadcasted_iota(jnp.int32, (_TR_OUT, _PROW), 1)
        r_local = (512 * (w // 128) + 128 * (col // 32) + w % 128)
        valid = (i * _TR_IN + r_local) < 1000000
        up_ref[...] = jnp.where(valid, pu, 0.0)
        ip_ref[...] = jnp.where(valid, pi, 0.0)


def _pack_call(uet, iet):
    return pl.pallas_call(
        _pack_body,
        grid=(_TR_GRID,),
        in_specs=[
            pl.BlockSpec((EMBED_DIM, _TR_IN), lambda i: (0, i)),
            pl.BlockSpec((EMBED_DIM, _TR_IN), lambda i: (0, i)),
        ],
        out_specs=(pl.BlockSpec((_TR_OUT, _PROW), lambda i: (i, 0)),
                   pl.BlockSpec((_TR_OUT, _PROW), lambda i: (i, 0))),
        out_shape=(jax.ShapeDtypeStruct((_NQ, _PROW), jnp.float32),
                   jax.ShapeDtypeStruct((_NQ, _PROW), jnp.float32)),
    )(uet, iet)


_MLP_BLK = 2048


def _mlp_body(yu_ref, yi_ref, ohu_ref, ohi_ref, mu_ref, mi_ref, b1_ref,
              w2_ref, b2_ref, out_ref):
    h = jnp.zeros((_MLP_BLK, HIDDEN_DIM), dtype=jnp.float32)
    yu = yu_ref[...]
    yi = yi_ref[...]
    for a in range(_PACK):
        hu = jnp.dot(yu, mu_ref[a], preferred_element_type=jnp.float32)
        hi = jnp.dot(yi, mi_ref[a], preferred_element_type=jnp.float32)
        h = h + hu * ohu_ref[:, a:a + 1] + hi * ohi_ref[:, a:a + 1]
    h = jnp.maximum(h + b1_ref[...], 0.0)
    y = jnp.sum(h * w2_ref[...], axis=1) + b2_ref[0, 0]
    out_ref[...] = jax.nn.sigmoid(y)


def _mlp_call(yu, yi, ohu, ohi, mu, mi, b1, w2, b2):
    grid = BATCH // _MLP_BLK
    return pl.pallas_call(
        _mlp_body,
        grid=(grid,),
        in_specs=[
            pl.BlockSpec((_MLP_BLK, _PROW), lambda i: (i, 0)),
            pl.BlockSpec((_MLP_BLK, _PROW), lambda i: (i, 0)),
            pl.BlockSpec((_MLP_BLK, _PACK), lambda i: (i, 0)),
            pl.BlockSpec((_MLP_BLK, _PACK), lambda i: (i, 0)),
            pl.BlockSpec((_PACK, _PROW, HIDDEN_DIM), lambda i: (0, 0, 0)),
            pl.BlockSpec((_PACK, _PROW, HIDDEN_DIM), lambda i: (0, 0, 0)),
            pl.BlockSpec((1, HIDDEN_DIM), lambda i: (0, 0)),
            pl.BlockSpec((1, HIDDEN_DIM), lambda i: (0, 0)),
            pl.BlockSpec((1, 1), lambda i: (0, 0)),
        ],
        out_specs=pl.BlockSpec((_MLP_BLK,), lambda i: (i,)),
        out_shape=jax.ShapeDtypeStruct((BATCH,), jnp.float32),
    )(yu, yi, ohu, ohi, mu, mi, b1, w2, b2)


def _shifted_weights(w_half):
    mats = []
    for a in range(_PACK):
        m = jnp.zeros((_PROW, HIDDEN_DIM), dtype=jnp.float32)
        m = lax.dynamic_update_slice(m, w_half, (a * EMBED_DIM, 0))
        mats.append(m)
    return jnp.stack(mats)


def kernel(user_ids, item_ids, user_emb, item_emb, W1, b1, W2, b2):
    uet = user_emb.T
    iet = item_emb.T
    up, ip = _pack_call(uet, iet)
    qu = jnp.bitwise_or(jnp.left_shift(jnp.right_shift(user_ids, 9), 7),
                        jnp.bitwise_and(user_ids, 127))
    qi = jnp.bitwise_or(jnp.left_shift(jnp.right_shift(item_ids, 9), 7),
                        jnp.bitwise_and(item_ids, 127))
    yu, yi = _sc_gather()(qu, qi, up, ip)
    au = jnp.bitwise_and(jnp.right_shift(user_ids, 7), _PACK - 1)
    ai = jnp.bitwise_and(jnp.right_shift(item_ids, 7), _PACK - 1)
    ohu = jax.nn.one_hot(au, _PACK, dtype=jnp.float32)
    ohi = jax.nn.one_hot(ai, _PACK, dtype=jnp.float32)
    mu = _shifted_weights(W1[:, :EMBED_DIM].T)
    mi = _shifted_weights(W1[:, EMBED_DIM:].T)
    b1r = b1.reshape(1, HIDDEN_DIM)
    w2r = W2.reshape(1, HIDDEN_DIM)
    b2r = b2.reshape(1, 1)
    return _mlp_call(yu, yi, ohu, ohi, mu, mi, b1r, w2r, b2r)
